# baseline (speedup 1.0000x reference)
import jax
import jax.numpy as jnp
from jax import lax
from jax.experimental import pallas as pl
from jax.experimental.pallas import tpu as pltpu
from jax.experimental.pallas import tpu_sc as plsc

MAX_POSITIONS = 1000000
EMBED_DIM = 32
BATCH = 16384
HIST = 200

NW = 32
NB = BATCH // 128
NA = HIST // 8
ITEMS = NA * NB
ITEMS_W = ITEMS // NW
L = 16


def _transpose_half(rows, tile, r0):
    lanes = lax.iota(jnp.int32, L)
    zeros = jnp.zeros((L,), jnp.int32)

    @plsc.parallel_loop(0, 256, unroll=16)
    def body(k):
        jh = k >> 5
        e = k & 31
        idx_j = jh * L + lanes
        e_vec = jnp.full((L,), e, jnp.int32)
        g_vec = jnp.full((L,), e >> 3, jnp.int32)
        dst = (e & 7) * 128 + jh * L + lanes
        for r in range(4):
            v = plsc.load_gather(rows.at[r0 + r], [idx_j, e_vec])
            plsc.store_scatter(tile.at[r], [g_vec, zeros, dst], v)


def _sc_gather(idx_hbm, table_hbm, out_hbm, idx_v0, idx_v1, rows_v0,
               rows_v1, tile_a, tile_b, isem0, isem1, gsem0, gsem1,
               wsem_a, wsem_b):
    idx_v = [idx_v0, idx_v1]
    rows_v = [rows_v0, rows_v1]
    isem = [isem0, isem1]
    gsem = [gsem0, gsem1]

    wid = lax.axis_index("s") * 2 + lax.axis_index("c")
    m0 = wid * ITEMS_W

    def start_idx(m, s):
        M = m0 + m
        pltpu.async_copy(
            idx_hbm.at[pl.ds(M // NB, 1), pl.ds(M % NB, 1)],
            idx_v[s], isem[s])

    def wait_idx(s):
        pltpu.make_async_copy(
            idx_hbm.at[pl.ds(0, 1), pl.ds(0, 1)], idx_v[s], isem[s]).wait()

    def fire_gathers(s):
        for r in range(8):
            pltpu.async_copy(
                table_hbm.at[idx_v[s].at[0, 0, r]], rows_v[s].at[r],
                gsem[s])

    def wait_gathers(s):
        for r in range(8):
            pltpu.make_async_copy(
                table_hbm.at[idx_v[s].at[0, 0, r]], rows_v[s].at[r],
                gsem[s]).wait()

    def out_slice(m, half):
        M = m0 + m
        return out_hbm.at[pl.ds((M // NB) * 8 + 4 * half, 4), :,
                          pl.ds(M % NB, 1), :]

    def wait_write(tile, sem):
        pltpu.make_async_copy(tile, out_slice(0, 0), sem).wait()

    def retire(m, s):
        wait_write(tile_a, wsem_a)
        wait_write(tile_b, wsem_b)
        _transpose_half(rows_v[s], tile_a, 0)
        pltpu.async_copy(tile_a, out_slice(m, 0), wsem_a)
        _transpose_half(rows_v[s], tile_b, 4)
        pltpu.async_copy(tile_b, out_slice(m, 1), wsem_b)

    def step(m, s, refill=True):
        o = 1 - s
        wait_idx(s)
        fire_gathers(s)
        wait_gathers(o)
        if refill:
            start_idx(m + 1, o)
        retire(m - 1, o)

    start_idx(0, 0)
    start_idx(1, 1)
    pltpu.async_copy(out_slice(0, 0), tile_a, wsem_a)
    pltpu.async_copy(out_slice(0, 1), tile_b, wsem_b)
    wait_idx(0)
    fire_gathers(0)

    def body(g, carry):
        m = 2 * g + 1
        step(m, 1)
        step(m + 1, 0)
        return carry

    lax.fori_loop(0, (ITEMS_W - 2) // 2, body, 0)

    step(99, 1, refill=False)
    wait_gathers(1)
    retire(99, 1)
    wait_write(tile_a, wsem_a)
    wait_write(tile_b, wsem_b)


@jax.jit
def _lookup(idx4, table):
    mesh = plsc.VectorSubcoreMesh(core_axis_name="c", subcore_axis_name="s")
    f = pl.kernel(
        _sc_gather,
        out_type=jax.ShapeDtypeStruct(
            (HIST, EMBED_DIM // 8, BATCH // 128, 1024), jnp.float32),
        mesh=mesh,
        scratch_types=(
            [pltpu.VMEM((1, 1, 8, 128), jnp.int32) for _ in range(2)]
            + [pltpu.VMEM((8, 128, EMBED_DIM), jnp.float32)
               for _ in range(2)]
            + [pltpu.VMEM((4, 4, 1, 1024), jnp.float32)
               for _ in range(2)]
            + [pltpu.SemaphoreType.DMA for _ in range(6)]
        ),
        compiler_params=pltpu.CompilerParams(
            use_tc_tiling_on_sc=False, needs_layout_passes=False),
    )
    return f(idx4, table)


def kernel(inputs, embeddings):
    idx4 = (inputs.astype(jnp.int32).T
            .reshape(NA, 8, NB, 128).transpose(0, 2, 1, 3))
    out = _lookup(idx4, embeddings)
    return (out.reshape(HIST, EMBED_DIM // 8, NB, 8, 128)
            .transpose(2, 4, 0, 1, 3).reshape(BATCH, HIST, EMBED_DIM))


# speedup vs baseline: 2.3552x; 2.3552x over previous
import jax
import jax.numpy as jnp
from jax import lax
from jax.experimental import pallas as pl
from jax.experimental.pallas import tpu as pltpu
from jax.experimental.pallas import tpu_sc as plsc

MAX_POSITIONS = 1000000
EMBED_DIM = 32
BATCH = 16384
HIST = 200

NW = 32
NB = BATCH // 128
NA = HIST // 8
ITEMS = NA * NB
ITEMS_W = ITEMS // NW
L = 16


def _transpose_half(rows, tile, r0):
    lanes = lax.iota(jnp.int32, L)
    zeros = jnp.zeros((L,), jnp.int32)

    @plsc.parallel_loop(0, 256, unroll=8)
    def body(k):
        j = k >> 1
        e0 = (k & 1) * L
        j_vec = jnp.full((L,), j, jnp.int32)
        e_vec = e0 + lanes
        g_vec = e_vec >> 3
        i_vec = e_vec & 7
        dst_j = jnp.full((L,), j, jnp.int32)
        for r in range(4):
            v = plsc.load_gather(rows.at[r0 + r], [j_vec, e_vec])
            plsc.store_scatter(tile.at[r], [g_vec, zeros, i_vec, dst_j], v)


def _sc_gather(idx_hbm, table_hbm, out_hbm, idx_v0, idx_v1, rows_v0,
               rows_v1, tile_a, tile_b, isem0, isem1, gsem0, gsem1,
               wsem_a, wsem_b):
    idx_v = [idx_v0, idx_v1]
    rows_v = [rows_v0, rows_v1]
    isem = [isem0, isem1]
    gsem = [gsem0, gsem1]

    wid = lax.axis_index("s") * 2 + lax.axis_index("c")
    m0 = wid * ITEMS_W

    def start_idx(m, s):
        M = m0 + m
        pltpu.async_copy(
            idx_hbm.at[pl.ds(M // NB, 1), pl.ds(M % NB, 1)],
            idx_v[s], isem[s])

    def wait_idx(s):
        pltpu.make_async_copy(
            idx_hbm.at[pl.ds(0, 1), pl.ds(0, 1)], idx_v[s], isem[s]).wait()

    def fire_gathers(s):
        for r in range(8):
            pltpu.async_copy(
                table_hbm.at[idx_v[s].at[0, 0, r]], rows_v[s].at[r],
                gsem[s])

    def wait_gathers(s):
        for r in range(8):
            pltpu.make_async_copy(
                table_hbm.at[idx_v[s].at[0, 0, r]], rows_v[s].at[r],
                gsem[s]).wait()

    def out_slice(m, half):
        M = m0 + m
        return out_hbm.at[pl.ds((M // NB) * 8 + 4 * half, 4), :,
                          pl.ds(M % NB, 1), :, :]

    def tile_body(tile):
        return tile.at[:, :, :, :, pl.ds(0, 128)]

    def wait_write(tile, sem):
        pltpu.make_async_copy(tile_body(tile), out_slice(0, 0), sem).wait()

    def retire(m, s):
        wait_write(tile_a, wsem_a)
        wait_write(tile_b, wsem_b)
        _transpose_half(rows_v[s], tile_a, 0)
        pltpu.async_copy(tile_body(tile_a), out_slice(m, 0), wsem_a)
        _transpose_half(rows_v[s], tile_b, 4)
        pltpu.async_copy(tile_body(tile_b), out_slice(m, 1), wsem_b)

    def step(m, s, refill=True):
        o = 1 - s
        wait_idx(s)
        fire_gathers(s)
        wait_gathers(o)
        if refill:
            start_idx(m + 1, o)
        retire(m - 1, o)

    start_idx(0, 0)
    start_idx(1, 1)
    pltpu.async_copy(out_slice(0, 0), tile_body(tile_a), wsem_a)
    pltpu.async_copy(out_slice(0, 1), tile_body(tile_b), wsem_b)
    wait_idx(0)
    fire_gathers(0)

    def body(g, carry):
        m = 2 * g + 1
        step(m, 1)
        step(m + 1, 0)
        return carry

    lax.fori_loop(0, (ITEMS_W - 2) // 2, body, 0)

    step(99, 1, refill=False)
    wait_gathers(1)
    retire(99, 1)
    wait_write(tile_a, wsem_a)
    wait_write(tile_b, wsem_b)


@jax.jit
def _lookup(idx4, table):
    mesh = plsc.VectorSubcoreMesh(core_axis_name="c", subcore_axis_name="s")
    f = pl.kernel(
        _sc_gather,
        out_type=jax.ShapeDtypeStruct(
            (HIST, EMBED_DIM // 8, BATCH // 128, 8, 128), jnp.float32),
        mesh=mesh,
        scratch_types=(
            [pltpu.VMEM((1, 1, 8, 128), jnp.int32) for _ in range(2)]
            + [pltpu.VMEM((8, 128, EMBED_DIM), jnp.float32)
               for _ in range(2)]
            + [pltpu.VMEM((4, 4, 1, 8, 129), jnp.float32)
               for _ in range(2)]
            + [pltpu.SemaphoreType.DMA for _ in range(6)]
        ),
        compiler_params=pltpu.CompilerParams(
            use_tc_tiling_on_sc=False, needs_layout_passes=False),
    )
    return f(idx4, table)


def kernel(inputs, embeddings):
    idx4 = (inputs.astype(jnp.int32).T
            .reshape(NA, 8, NB, 128).transpose(0, 2, 1, 3))
    out = _lookup(idx4, embeddings)
    return (out.transpose(2, 4, 0, 1, 3)
            .reshape(BATCH, HIST, EMBED_DIM))
